# baseline (speedup 1.0000x reference)
import functools

import jax
import jax.numpy as jnp
from jax import lax
from jax.experimental import pallas as pl
from jax.experimental.pallas import tpu as pltpu
from jax.experimental.pallas import tpu_sc as plsc

NC, NS = 2, 16
NW = NC * NS
BATCH = 16384
BPW = BATCH // NW
NE = BPW * 3
NG = NE // 16
NELEM = 3000000

_MESH = plsc.VectorSubcoreMesh(
    core_axis_name="c", subcore_axis_name="s",
    num_cores=NC, num_subcores=NS,
)


@functools.partial(
    pl.kernel,
    out_type=jax.ShapeDtypeStruct((NW, NE), jnp.float32),
    mesh=_MESH,
    scratch_types=[
        pltpu.VMEM((BPW,), jnp.int32),
        pltpu.VMEM((NE,), jnp.float32),
        pltpu.SemaphoreType.DMA,
    ],
    compiler_params=pltpu.CompilerParams(needs_layout_passes=False),
)
def _gather_sc(flat_hbm, idx_hbm, out_hbm, idx_v, rows, sem):
    wid = lax.axis_index("s") * NC + lax.axis_index("c")
    pltpu.sync_copy(idx_hbm.at[wid], idx_v)
    lanes = lax.iota(jnp.int32, 16)
    copies = []
    for t in range(NG):
        p = lanes + (t * 16)
        r = p // 3
        c = p - r * 3
        vidx = plsc.load_gather(idx_v, [r])
        e = vidx * 3 + c
        copies.append(
            pltpu.async_copy(flat_hbm.at[e], rows.at[pl.ds(t * 16, 16)], sem)
        )
    for cp in copies:
        cp.wait()
    pltpu.sync_copy(rows, out_hbm.at[wid])


def kernel(view_correction, index):
    flat = view_correction.reshape(NELEM)
    idx = index.reshape(NW, BPW)
    out = _gather_sc(flat, idx)
    return out.reshape(BATCH, 3)


# speedup vs baseline: 36.1023x; 36.1023x over previous
import functools

import jax
import jax.numpy as jnp
from jax import lax
from jax.experimental import pallas as pl
from jax.experimental.pallas import tpu as pltpu
from jax.experimental.pallas import tpu_sc as plsc

NC, NS = 2, 16
NW = NC * NS
BATCH = 16384
BPW = BATCH // NW
NE = BPW * 3
NGI = BPW // 16
NGO = NE // 16

_MESH = plsc.VectorSubcoreMesh(
    core_axis_name="c", subcore_axis_name="s",
    num_cores=NC, num_subcores=NS,
)


@functools.partial(
    pl.kernel,
    out_type=jax.ShapeDtypeStruct((NW, NE), jnp.float32),
    mesh=_MESH,
    scratch_types=[
        pltpu.VMEM((BPW,), jnp.int32),
        pltpu.VMEM((3, BPW), jnp.float32),
        pltpu.VMEM((NE,), jnp.float32),
        pltpu.SemaphoreType.DMA,
    ],
    compiler_params=pltpu.CompilerParams(needs_layout_passes=False),
)
def _gather_sc(c0_hbm, c1_hbm, c2_hbm, idx_hbm, out_hbm,
               idx_v, colsT, rows, sem):
    wid = lax.axis_index("s") * NC + lax.axis_index("c")
    pltpu.sync_copy(idx_hbm.at[wid], idx_v)
    lanes = lax.iota(jnp.int32, 16)
    copies = []
    for t in range(NGI):
        v = idx_v[pl.ds(t * 16, 16)]
        for c, col in enumerate((c0_hbm, c1_hbm, c2_hbm)):
            copies.append(
                pltpu.async_copy(col.at[v], colsT.at[c, pl.ds(t * 16, 16)], sem)
            )
    for cp in copies:
        cp.wait()
    for t in range(NGO):
        p = lanes + (t * 16)
        r = p // 3
        c = p - r * 3
        rows[pl.ds(t * 16, 16)] = plsc.load_gather(colsT, [c, r])
    pltpu.sync_copy(rows, out_hbm.at[wid])


def kernel(view_correction, index):
    c0 = view_correction[:, 0]
    c1 = view_correction[:, 1]
    c2 = view_correction[:, 2]
    idx = index.reshape(NW, BPW)
    out = _gather_sc(c0, c1, c2, idx)
    return out.reshape(BATCH, 3)


# speedup vs baseline: 44.7166x; 1.2386x over previous
import functools

import jax
import jax.numpy as jnp
from jax import lax
from jax.experimental import pallas as pl
from jax.experimental.pallas import tpu as pltpu
from jax.experimental.pallas import tpu_sc as plsc

NC, NS = 2, 16
NW = NC * NS
BATCH = 16384
BPW = BATCH // NW
NE = BPW * 3
NG = NE // 16
NVIEWS = 1000000

_MESH = plsc.VectorSubcoreMesh(
    core_axis_name="c", subcore_axis_name="s",
    num_cores=NC, num_subcores=NS,
)


@functools.partial(
    pl.kernel,
    out_type=jax.ShapeDtypeStruct((NW, NE), jnp.float32),
    mesh=_MESH,
    scratch_types=[
        pltpu.VMEM((BPW,), jnp.int32),
        pltpu.VMEM((NE,), jnp.float32),
        pltpu.SemaphoreType.DMA,
    ],
    compiler_params=pltpu.CompilerParams(needs_layout_passes=False),
)
def _gather_sc(flat_hbm, idx_hbm, out_hbm, idx_v, rows, sem):
    wid = lax.axis_index("s") * NC + lax.axis_index("c")
    pltpu.sync_copy(idx_hbm.at[wid], idx_v)
    lanes = lax.iota(jnp.int32, 16)
    copies = []
    for t in range(NG):
        p = lanes + (t * 16)
        r = p // 3
        c = p - r * 3
        vidx = plsc.load_gather(idx_v, [r])
        e = c * NVIEWS + vidx
        copies.append(
            pltpu.async_copy(flat_hbm.at[e], rows.at[pl.ds(t * 16, 16)], sem)
        )
    for cp in copies:
        cp.wait()
    pltpu.sync_copy(rows, out_hbm.at[wid])


def kernel(view_correction, index):
    flat = view_correction.T.reshape(NVIEWS * 3)
    idx = index.reshape(NW, BPW)
    out = _gather_sc(flat, idx)
    return out.reshape(BATCH, 3)


# speedup vs baseline: 59.5603x; 1.3320x over previous
import functools

import jax
import jax.numpy as jnp
from jax import lax
from jax.experimental import pallas as pl
from jax.experimental.pallas import tpu as pltpu
from jax.experimental.pallas import tpu_sc as plsc

NC, NS = 2, 16
NW = NC * NS
BATCH = 16384
BPW = BATCH // NW
NG = BPW // 16
NVIEWS = 1000000

_MESH = plsc.VectorSubcoreMesh(
    core_axis_name="c", subcore_axis_name="s",
    num_cores=NC, num_subcores=NS,
)


@functools.partial(
    pl.kernel,
    out_type=jax.ShapeDtypeStruct((3, BATCH), jnp.float32),
    mesh=_MESH,
    scratch_types=[
        pltpu.VMEM((BPW,), jnp.int32),
        pltpu.VMEM((3, BPW), jnp.float32),
        pltpu.SemaphoreType.DMA,
    ],
    compiler_params=pltpu.CompilerParams(needs_layout_passes=False),
)
def _gather_sc(vt_hbm, idx_hbm, out_hbm, idx_v, colsT, sem):
    wid = lax.axis_index("s") * NC + lax.axis_index("c")
    pltpu.sync_copy(idx_hbm.at[pl.ds(wid * BPW, BPW)], idx_v)
    copies = []
    for t in range(NG):
        v = idx_v[pl.ds(t * 16, 16)]
        for c in range(3):
            copies.append(
                pltpu.async_copy(vt_hbm.at[v + (c * NVIEWS)],
                                 colsT.at[c, pl.ds(t * 16, 16)], sem)
            )
    for cp in copies:
        cp.wait()
    pltpu.sync_copy(colsT, out_hbm.at[:, pl.ds(wid * BPW, BPW)])


def kernel(view_correction, index):
    out = _gather_sc(view_correction.T.reshape(3 * NVIEWS), index)
    return out.T
